# Initial kernel scaffold; baseline (speedup 1.0000x reference)
import jax
import jax.numpy as jnp
from jax.experimental import pallas as pl


def kernel(part_features, altitude_idx, gamma, beta):
    raise NotImplementedError("write your pallas kernel here")


# speedup vs baseline: 1.8760x; 1.8760x over previous
import functools

import jax
import jax.numpy as jnp
from jax import lax
from jax.experimental import pallas as pl
from jax.experimental.pallas import tpu as pltpu
from jax.experimental.pallas import tpu_sc as plsc

B, K, D = 16384, 8, 256
L = 16

NC, NS = 2, 16
NW = NC * NS
ROWS_PER_W = B // NW
CH = 8
NCHUNK = ROWS_PER_W // CH
NBUF = 2
NPAIR = NCHUNK // NBUF

_mesh = plsc.VectorSubcoreMesh(core_axis_name="c", subcore_axis_name="s")


def _sc_body(pf_hbm, idx_hbm, gamma_hbm, beta_hbm, out_hbm,
             gamma_v, beta_v, idx_v,
             ibuf0, ibuf1, obuf0, obuf1,
             sin0, sin1, sout0, sout1):
    ibufs = (ibuf0, ibuf1)
    obufs = (obuf0, obuf1)
    sin = (sin0, sin1)
    sout = (sout0, sout1)

    wid = lax.axis_index("s") * NC + lax.axis_index("c")
    base = wid * ROWS_PER_W

    pltpu.sync_copy(gamma_hbm, gamma_v)
    pltpu.sync_copy(beta_hbm, beta_v)
    pltpu.sync_copy(idx_hbm.at[pl.ds(base, ROWS_PER_W)],
                    idx_v.at[pl.ds(0, ROWS_PER_W)])

    for p in range(NBUF):
        pltpu.make_async_copy(
            pf_hbm.at[pl.ds(base + p * CH, CH)], ibufs[p], sin[p]).start()

    def chunk_pair(i, carry):
        for p in range(NBUF):
            c = i * NBUF + p
            row0 = base + c * CH
            pltpu.make_async_copy(
                pf_hbm.at[pl.ds(row0, CH)], ibufs[p], sin[p]).wait()

            @pl.when(c >= NBUF)
            def _drain():
                pltpu.make_async_copy(
                    obufs[p], out_hbm.at[pl.ds(row0 - NBUF * CH, CH)],
                    sout[p]).wait()

            def row_body(r, _):
                av = idx_v[pl.ds(c * CH + r, L)]
                a = av[0]
                for j in range(D // L):
                    g = gamma_v[a, pl.ds(j * L, L)]
                    bt = beta_v[a, pl.ds(j * L, L)]
                    for k in range(K):
                        obufs[p][r, k, pl.ds(j * L, L)] = (
                            ibufs[p][r, k, pl.ds(j * L, L)] * g + bt)
                return 0

            lax.fori_loop(0, CH, row_body, 0, unroll=False)

            pltpu.make_async_copy(
                obufs[p], out_hbm.at[pl.ds(row0, CH)], sout[p]).start()

            @pl.when(c + NBUF < NCHUNK)
            def _prefetch():
                pltpu.make_async_copy(
                    pf_hbm.at[pl.ds(row0 + NBUF * CH, CH)], ibufs[p],
                    sin[p]).start()
        return carry

    lax.fori_loop(0, NPAIR, chunk_pair, 0, unroll=False)

    for p in range(NBUF):
        c = NCHUNK - NBUF + p
        pltpu.make_async_copy(
            obufs[p], out_hbm.at[pl.ds(base + c * CH, CH)], sout[p]).wait()


_film_sc = functools.partial(
    pl.kernel,
    mesh=_mesh,
    out_type=jax.ShapeDtypeStruct((B, K, D), jnp.float32),
    scratch_types=[
        pltpu.VMEM((4, D), jnp.float32),
        pltpu.VMEM((4, D), jnp.float32),
        pltpu.VMEM((ROWS_PER_W + L,), jnp.int32),
        pltpu.VMEM((CH, K, D), jnp.float32),
        pltpu.VMEM((CH, K, D), jnp.float32),
        pltpu.VMEM((CH, K, D), jnp.float32),
        pltpu.VMEM((CH, K, D), jnp.float32),
        pltpu.SemaphoreType.DMA,
        pltpu.SemaphoreType.DMA,
        pltpu.SemaphoreType.DMA,
        pltpu.SemaphoreType.DMA,
    ],
)(_sc_body)


def kernel(part_features, altitude_idx, gamma, beta):
    idx32 = altitude_idx.astype(jnp.int32)
    return _film_sc(part_features, idx32, gamma, beta)
